# baseline (speedup 1.0000x reference)
import functools
import jax
import jax.numpy as jnp
from jax import lax
from jax.experimental import pallas as pl
from jax.experimental.pallas import tpu as pltpu
from jax.experimental.pallas import tpu_sc as plsc

_D = 256
_N_LEVELS = 16

_TC_ROWS = 0


def _tc_body(scal_ref, x_ref, ab_ref, o_ref):
    x = x_ref[...]
    n2 = jnp.sum(x * x, axis=1, keepdims=True)
    norm = jnp.maximum(jnp.sqrt(n2), 1e-8)
    inv = 1.0 / norm

    lane = lax.broadcasted_iota(jnp.int32, (1, _D), 1)
    even = (lane % 2) == 0

    a = ab_ref[0:1, :]
    b = ab_ref[1:2, :]

    xs = jnp.where(even, jnp.roll(x, -1, axis=1), jnp.roll(x, 1, axis=1))
    f = (a * x + b * xs) * inv

    q = jnp.full(f.shape, scal_ref[0], dtype=jnp.float32)
    for k in range(_N_LEVELS - 1):
        q = q + jnp.where(f > scal_ref[1 + k], scal_ref[16 + k], 0.0)

    qs = jnp.where(even, jnp.roll(q, -1, axis=1), jnp.roll(q, 1, axis=1))
    o_ref[...] = (a * q + b * qs) * norm


def _tc_quant(x, ab, scal, bm):
    B = x.shape[0]
    grid = (B // bm,)
    return pl.pallas_call(
        _tc_body,
        grid=grid,
        in_specs=[
            pl.BlockSpec(memory_space=pltpu.SMEM),
            pl.BlockSpec((bm, _D), lambda i: (i, 0)),
            pl.BlockSpec((2, _D), lambda i: (0, 0)),
        ],
        out_specs=pl.BlockSpec((bm, _D), lambda i: (i, 0)),
        out_shape=jax.ShapeDtypeStruct((B, _D), jnp.float32),
    )(scal, x, ab)


_CH = 32

_GDN = lax.GatherDimensionNumbers(
    offset_dims=(), collapsed_slice_dims=(0,), start_index_map=(0,))


def _take16(v, idx):
    return lax.gather(v, idx[:, None], _GDN, slice_sizes=(1,),
                      mode=lax.GatherScatterMode.PROMISE_IN_BOUNDS)


def _sc_quant(x_flat, abt, n_rows):
    info = plsc.get_sparse_core_info()
    NC, NS = info.num_cores, info.num_subcores
    NW = NC * NS
    rows_w = n_rows // NW
    n_ch = rows_w // _CH
    mesh = plsc.VectorSubcoreMesh(core_axis_name="c", subcore_axis_name="s")

    @functools.partial(
        pl.kernel,
        mesh=mesh,
        out_type=jax.ShapeDtypeStruct((n_rows * _D,), jnp.float32),
        scratch_types=[
            pltpu.VMEM((_CH * _D,), jnp.float32),
            pltpu.VMEM((_CH * _D,), jnp.float32),
            pltpu.VMEM((544,), jnp.float32),
        ],
    )
    def k(x_hbm, abt_hbm, o_hbm, xbuf, obuf, cbuf):
        wid = lax.axis_index("s") * NC + lax.axis_index("c")
        base = wid * rows_w
        pltpu.sync_copy(abt_hbm, cbuf)
        lane = lax.iota(jnp.int32, 16)
        swap = lane ^ 1
        mids_v = cbuf[pl.ds(512, 16)]
        cents_v = cbuf[pl.ds(528, 16)]
        m7 = _take16(mids_v, jnp.full((16,), 7, jnp.int32))

        def chunk_body(g, _carry):
            start = (base + g * _CH) * _D
            pltpu.sync_copy(x_hbm.at[pl.ds(start, _CH * _D)], xbuf)

            def row_body(r, _c2):
                p = r * _D
                accs = [jnp.zeros((16,), jnp.float32) for _ in range(4)]
                for t in range(16):
                    v = xbuf[pl.ds(p + t * 16, 16)]
                    accs[t % 4] = accs[t % 4] + v * v
                acc = (accs[0] + accs[1]) + (accs[2] + accs[3])
                for sh in (8, 4, 2, 1):
                    acc = acc + _take16(acc, lane ^ sh)
                sv = jnp.maximum(acc, 1e-30)
                i32 = lax.bitcast_convert_type(sv, jnp.int32)
                y = lax.bitcast_convert_type(
                    jnp.int32(0x5F3759DF) - (i32 >> 1), jnp.float32)
                for _ in range(3):
                    y = y * (1.5 - 0.5 * sv * y * y)
                norm = jnp.maximum(sv * y, 1e-8)
                inv = 1.0 / norm
                for g4 in range(4):
                    ts = [4 * g4 + j for j in range(4)]
                    vl = [xbuf[pl.ds(p + t * 16, 16)] for t in ts]
                    al = [cbuf[pl.ds(t * 16, 16)] for t in ts]
                    bl = [cbuf[pl.ds(256 + t * 16, 16)] for t in ts]
                    fl = [(al[j] * vl[j] + bl[j] * _take16(vl[j], swap)) * inv
                          for j in range(4)]
                    il = [jnp.where(fl[j] > m7, 8, 0) for j in range(4)]
                    for bit in (4, 2, 1):
                        tm = [_take16(mids_v, il[j] | (bit - 1))
                              for j in range(4)]
                        il = [il[j] | jnp.where(fl[j] > tm[j], bit, 0)
                              for j in range(4)]
                    ql = [_take16(cents_v, il[j]) for j in range(4)]
                    ol = [(al[j] * ql[j] + bl[j] * _take16(ql[j], swap)) * norm
                          for j in range(4)]
                    for j in range(4):
                        obuf[pl.ds(p + ts[j] * 16, 16)] = ol[j]
                return 0

            lax.fori_loop(0, _CH, row_body, 0)
            pltpu.sync_copy(obuf, o_hbm.at[pl.ds(start, _CH * _D)])
            return 0

        lax.fori_loop(0, n_ch, chunk_body, 0)

    return k(x_flat, abt)


@jax.jit
def kernel(x, rot2, centroids):
    c = rot2[:, 0]
    s = rot2[:, 1]
    a = jnp.repeat(c, 2)
    b = jnp.stack([-s, s], axis=-1).reshape(-1)
    mids = 0.5 * (centroids[1:] + centroids[:-1])
    dlt = centroids[1:] - centroids[:-1]

    outs = []
    if _TC_ROWS > 0:
        ab = jnp.stack([a, b], axis=0)
        scal = jnp.concatenate(
            [centroids[0:1], mids, dlt, jnp.zeros((1,), jnp.float32)])
        outs.append(_tc_quant(x[:_TC_ROWS], ab, scal, bm=512))

    n_sc = x.shape[0] - _TC_ROWS
    if n_sc > 0:
        abt = jnp.concatenate(
            [a, b, mids, jnp.zeros((1,), jnp.float32), centroids])
        sc_out = _sc_quant(
            jnp.reshape(x[_TC_ROWS:], (-1,)), abt, n_sc)
        outs.append(jnp.reshape(sc_out, (n_sc, _D)))

    if len(outs) == 1:
        return outs[0]
    return jnp.concatenate(outs, axis=0)


# speedup vs baseline: 1.6235x; 1.6235x over previous
import functools
import jax
import jax.numpy as jnp
from jax import lax
from jax.experimental import pallas as pl
from jax.experimental.pallas import tpu as pltpu
from jax.experimental.pallas import tpu_sc as plsc

_D = 256
_N_LEVELS = 16

_TC_ROWS = 7168


def _tc_body(scal_ref, x_ref, ab_ref, o_ref):
    x = x_ref[...]
    n2 = jnp.sum(x * x, axis=1, keepdims=True)
    norm = jnp.maximum(jnp.sqrt(n2), 1e-8)
    inv = 1.0 / norm

    lane = lax.broadcasted_iota(jnp.int32, (1, _D), 1)
    even = (lane % 2) == 0

    a = ab_ref[0:1, :]
    b = ab_ref[1:2, :]

    xs = jnp.where(even, jnp.roll(x, -1, axis=1), jnp.roll(x, 1, axis=1))
    f = (a * x + b * xs) * inv

    q = jnp.full(f.shape, scal_ref[0], dtype=jnp.float32)
    for k in range(_N_LEVELS - 1):
        q = q + jnp.where(f > scal_ref[1 + k], scal_ref[16 + k], 0.0)

    qs = jnp.where(even, jnp.roll(q, -1, axis=1), jnp.roll(q, 1, axis=1))
    o_ref[...] = (a * q + b * qs) * norm


def _tc_quant(x, ab, scal, bm):
    B = x.shape[0]
    grid = (B // bm,)
    return pl.pallas_call(
        _tc_body,
        grid=grid,
        in_specs=[
            pl.BlockSpec(memory_space=pltpu.SMEM),
            pl.BlockSpec((bm, _D), lambda i: (i, 0)),
            pl.BlockSpec((2, _D), lambda i: (0, 0)),
        ],
        out_specs=pl.BlockSpec((bm, _D), lambda i: (i, 0)),
        out_shape=jax.ShapeDtypeStruct((B, _D), jnp.float32),
    )(scal, x, ab)


_CH = 32

_GDN = lax.GatherDimensionNumbers(
    offset_dims=(), collapsed_slice_dims=(0,), start_index_map=(0,))


def _take16(v, idx):
    return lax.gather(v, idx[:, None], _GDN, slice_sizes=(1,),
                      mode=lax.GatherScatterMode.PROMISE_IN_BOUNDS)


def _sc_quant(x_flat, abt, n_rows):
    info = plsc.get_sparse_core_info()
    NC, NS = info.num_cores, info.num_subcores
    NW = NC * NS
    rows_w = n_rows // NW
    n_ch = rows_w // _CH
    mesh = plsc.VectorSubcoreMesh(core_axis_name="c", subcore_axis_name="s")

    @functools.partial(
        pl.kernel,
        mesh=mesh,
        out_type=jax.ShapeDtypeStruct((n_rows * _D,), jnp.float32),
        scratch_types=[
            pltpu.VMEM((_CH * _D,), jnp.float32),
            pltpu.VMEM((_CH * _D,), jnp.float32),
            pltpu.VMEM((544,), jnp.float32),
        ],
    )
    def k(x_hbm, abt_hbm, o_hbm, xbuf, obuf, cbuf):
        wid = lax.axis_index("s") * NC + lax.axis_index("c")
        base = wid * rows_w
        pltpu.sync_copy(abt_hbm, cbuf)
        lane = lax.iota(jnp.int32, 16)
        swap = lane ^ 1
        mids_v = cbuf[pl.ds(512, 16)]
        cents_v = cbuf[pl.ds(528, 16)]
        m7 = _take16(mids_v, jnp.full((16,), 7, jnp.int32))

        def chunk_body(g, _carry):
            start = (base + g * _CH) * _D
            pltpu.sync_copy(x_hbm.at[pl.ds(start, _CH * _D)], xbuf)

            def row_body(r, _c2):
                p = r * _D
                accs = [jnp.zeros((16,), jnp.float32) for _ in range(4)]
                for t in range(16):
                    v = xbuf[pl.ds(p + t * 16, 16)]
                    accs[t % 4] = accs[t % 4] + v * v
                acc = (accs[0] + accs[1]) + (accs[2] + accs[3])
                for sh in (8, 4, 2, 1):
                    acc = acc + _take16(acc, lane ^ sh)
                sv = jnp.maximum(acc, 1e-30)
                i32 = lax.bitcast_convert_type(sv, jnp.int32)
                y = lax.bitcast_convert_type(
                    jnp.int32(0x5F3759DF) - (i32 >> 1), jnp.float32)
                for _ in range(3):
                    y = y * (1.5 - 0.5 * sv * y * y)
                norm = jnp.maximum(sv * y, 1e-8)
                inv = 1.0 / norm
                for g4 in range(4):
                    ts = [4 * g4 + j for j in range(4)]
                    vl = [xbuf[pl.ds(p + t * 16, 16)] for t in ts]
                    al = [cbuf[pl.ds(t * 16, 16)] for t in ts]
                    bl = [cbuf[pl.ds(256 + t * 16, 16)] for t in ts]
                    fl = [(al[j] * vl[j] + bl[j] * _take16(vl[j], swap)) * inv
                          for j in range(4)]
                    il = [jnp.where(fl[j] > m7, 8, 0) for j in range(4)]
                    for bit in (4, 2, 1):
                        tm = [_take16(mids_v, il[j] | (bit - 1))
                              for j in range(4)]
                        il = [il[j] | jnp.where(fl[j] > tm[j], bit, 0)
                              for j in range(4)]
                    ql = [_take16(cents_v, il[j]) for j in range(4)]
                    ol = [(al[j] * ql[j] + bl[j] * _take16(ql[j], swap)) * norm
                          for j in range(4)]
                    for j in range(4):
                        obuf[pl.ds(p + ts[j] * 16, 16)] = ol[j]
                return 0

            lax.fori_loop(0, _CH, row_body, 0)
            pltpu.sync_copy(obuf, o_hbm.at[pl.ds(start, _CH * _D)])
            return 0

        lax.fori_loop(0, n_ch, chunk_body, 0)

    return k(x_flat, abt)


@jax.jit
def kernel(x, rot2, centroids):
    c = rot2[:, 0]
    s = rot2[:, 1]
    a = jnp.repeat(c, 2)
    b = jnp.stack([-s, s], axis=-1).reshape(-1)
    mids = 0.5 * (centroids[1:] + centroids[:-1])
    dlt = centroids[1:] - centroids[:-1]

    outs = []
    if _TC_ROWS > 0:
        ab = jnp.stack([a, b], axis=0)
        scal = jnp.concatenate(
            [centroids[0:1], mids, dlt, jnp.zeros((1,), jnp.float32)])
        outs.append(_tc_quant(x[:_TC_ROWS], ab, scal, bm=512))

    n_sc = x.shape[0] - _TC_ROWS
    if n_sc > 0:
        abt = jnp.concatenate(
            [a, b, mids, jnp.zeros((1,), jnp.float32), centroids])
        sc_out = _sc_quant(
            jnp.reshape(x[_TC_ROWS:], (-1,)), abt, n_sc)
        outs.append(jnp.reshape(sc_out, (n_sc, _D)))

    if len(outs) == 1:
        return outs[0]
    return jnp.concatenate(outs, axis=0)


# speedup vs baseline: 1.6500x; 1.0163x over previous
import functools
import jax
import jax.numpy as jnp
from jax import lax
from jax.experimental import pallas as pl
from jax.experimental.pallas import tpu as pltpu
from jax.experimental.pallas import tpu_sc as plsc

_D = 256
_N_LEVELS = 16

_TC_ROWS = 7168


def _tc_body(scal_ref, x_ref, ab_ref, o_ref):
    x = x_ref[...]
    n2 = jnp.sum(x * x, axis=1, keepdims=True)
    norm = jnp.maximum(jnp.sqrt(n2), 1e-8)
    inv = 1.0 / norm

    lane = lax.broadcasted_iota(jnp.int32, (1, _D), 1)
    even = (lane % 2) == 0

    a = ab_ref[0:1, :]
    b = ab_ref[1:2, :]

    xs = jnp.where(even, jnp.roll(x, -1, axis=1), jnp.roll(x, 1, axis=1))
    f = (a * x + b * xs) * inv

    q = jnp.full(f.shape, scal_ref[0], dtype=jnp.float32)
    for k in range(_N_LEVELS - 1):
        q = q + jnp.where(f > scal_ref[1 + k], scal_ref[16 + k], 0.0)

    qs = jnp.where(even, jnp.roll(q, -1, axis=1), jnp.roll(q, 1, axis=1))
    o_ref[...] = (a * q + b * qs) * norm


def _tc_quant(x, ab, scal, bm, n_rows=None):
    B = x.shape[0]
    if n_rows is None:
        n_rows = B
    grid = (n_rows // bm,)
    return pl.pallas_call(
        _tc_body,
        grid=grid,
        in_specs=[
            pl.BlockSpec(memory_space=pltpu.SMEM),
            pl.BlockSpec((bm, _D), lambda i: (i, 0)),
            pl.BlockSpec((2, _D), lambda i: (0, 0)),
        ],
        out_specs=pl.BlockSpec((bm, _D), lambda i: (i, 0)),
        out_shape=jax.ShapeDtypeStruct((n_rows, _D), jnp.float32),
    )(scal, x, ab)


_CH = 32

_GDN = lax.GatherDimensionNumbers(
    offset_dims=(), collapsed_slice_dims=(0,), start_index_map=(0,))


def _take16(v, idx):
    return lax.gather(v, idx[:, None], _GDN, slice_sizes=(1,),
                      mode=lax.GatherScatterMode.PROMISE_IN_BOUNDS)


def _sc_quant(x_flat, abt, n_rows, row0):
    info = plsc.get_sparse_core_info()
    NC, NS = info.num_cores, info.num_subcores
    NW = NC * NS
    rows_w = n_rows // NW
    n_ch = rows_w // _CH
    mesh = plsc.VectorSubcoreMesh(core_axis_name="c", subcore_axis_name="s")

    @functools.partial(
        pl.kernel,
        mesh=mesh,
        out_type=jax.ShapeDtypeStruct((n_rows * _D,), jnp.float32),
        scratch_types=[
            pltpu.VMEM((_CH * _D,), jnp.float32),
            pltpu.VMEM((_CH * _D,), jnp.float32),
            pltpu.VMEM((544,), jnp.float32),
        ],
    )
    def k(x_hbm, abt_hbm, o_hbm, xbuf, obuf, cbuf):
        wid = lax.axis_index("s") * NC + lax.axis_index("c")
        base = wid * rows_w
        pltpu.sync_copy(abt_hbm, cbuf)
        lane = lax.iota(jnp.int32, 16)
        swap = lane ^ 1
        mids_v = cbuf[pl.ds(512, 16)]
        cents_v = cbuf[pl.ds(528, 16)]
        m7 = _take16(mids_v, jnp.full((16,), 7, jnp.int32))

        def chunk_body(g, _carry):
            start = (base + g * _CH) * _D
            pltpu.sync_copy(x_hbm.at[pl.ds(row0 * _D + start, _CH * _D)], xbuf)

            def row_body(r, _c2):
                p = r * _D
                accs = [jnp.zeros((16,), jnp.float32) for _ in range(4)]
                for t in range(16):
                    v = xbuf[pl.ds(p + t * 16, 16)]
                    accs[t % 4] = accs[t % 4] + v * v
                acc = (accs[0] + accs[1]) + (accs[2] + accs[3])
                for sh in (8, 4, 2, 1):
                    acc = acc + _take16(acc, lane ^ sh)
                sv = jnp.maximum(acc, 1e-30)
                i32 = lax.bitcast_convert_type(sv, jnp.int32)
                y = lax.bitcast_convert_type(
                    jnp.int32(0x5F3759DF) - (i32 >> 1), jnp.float32)
                for _ in range(3):
                    y = y * (1.5 - 0.5 * sv * y * y)
                norm = jnp.maximum(sv * y, 1e-8)
                inv = 1.0 / norm
                for g4 in range(4):
                    ts = [4 * g4 + j for j in range(4)]
                    vl = [xbuf[pl.ds(p + t * 16, 16)] for t in ts]
                    al = [cbuf[pl.ds(t * 16, 16)] for t in ts]
                    bl = [cbuf[pl.ds(256 + t * 16, 16)] for t in ts]
                    fl = [(al[j] * vl[j] + bl[j] * _take16(vl[j], swap)) * inv
                          for j in range(4)]
                    il = [jnp.where(fl[j] > m7, 8, 0) for j in range(4)]
                    for bit in (4, 2, 1):
                        tm = [_take16(mids_v, il[j] | (bit - 1))
                              for j in range(4)]
                        il = [il[j] | jnp.where(fl[j] > tm[j], bit, 0)
                              for j in range(4)]
                    ql = [_take16(cents_v, il[j]) for j in range(4)]
                    ol = [(al[j] * ql[j] + bl[j] * _take16(ql[j], swap)) * norm
                          for j in range(4)]
                    for j in range(4):
                        obuf[pl.ds(p + ts[j] * 16, 16)] = ol[j]
                return 0

            lax.fori_loop(0, _CH, row_body, 0)
            pltpu.sync_copy(obuf, o_hbm.at[pl.ds(start, _CH * _D)])
            return 0

        lax.fori_loop(0, n_ch, chunk_body, 0)

    return k(x_flat, abt)


@jax.jit
def kernel(x, rot2, centroids):
    c = rot2[:, 0]
    s = rot2[:, 1]
    a = jnp.repeat(c, 2)
    b = jnp.stack([-s, s], axis=-1).reshape(-1)
    mids = 0.5 * (centroids[1:] + centroids[:-1])
    dlt = centroids[1:] - centroids[:-1]

    outs = []
    if _TC_ROWS > 0:
        ab = jnp.stack([a, b], axis=0)
        scal = jnp.concatenate(
            [centroids[0:1], mids, dlt, jnp.zeros((1,), jnp.float32)])
        outs.append(_tc_quant(x, ab, scal, bm=512, n_rows=_TC_ROWS))

    n_sc = x.shape[0] - _TC_ROWS
    if n_sc > 0:
        abt = jnp.concatenate(
            [a, b, mids, jnp.zeros((1,), jnp.float32), centroids])
        sc_out = _sc_quant(
            jnp.reshape(x, (-1,)), abt, n_sc, _TC_ROWS)
        outs.append(jnp.reshape(sc_out, (n_sc, _D)))

    if len(outs) == 1:
        return outs[0]
    return jnp.concatenate(outs, axis=0)


# speedup vs baseline: 2.5927x; 1.5713x over previous
import functools
import jax
import jax.numpy as jnp
from jax import lax
from jax.experimental import pallas as pl
from jax.experimental.pallas import tpu as pltpu
from jax.experimental.pallas import tpu_sc as plsc

_D = 256

_TC_ROWS = 7168
_TC_BM = 1024


def _tc_body(cents_ref, x_ref, rf_ref, o_ref):
    x = x_ref[...]
    n2 = jnp.sum(x * x, axis=1, keepdims=True)
    norm = jnp.maximum(jnp.sqrt(n2), 1e-8)
    inv = 1.0 / norm

    lane = lax.broadcasted_iota(jnp.int32, (1, _D), 1)
    even = (lane % 2) == 0

    flat = rf_ref[...]
    fsw = jnp.where(even, jnp.roll(flat, -1, axis=1),
                    jnp.roll(flat, 1, axis=1))
    a = jnp.where(even, flat, fsw)
    b = jnp.where(even, -fsw, flat)

    xs = jnp.where(even, jnp.roll(x, -1, axis=1), jnp.roll(x, 1, axis=1))
    f = (a * x + b * xs) * inv

    def ctr(k):
        return cents_ref[k]

    def m(k):
        return 0.5 * (cents_ref[k] + cents_ref[k + 1])

    def sel(cond, hi, lo):
        return jnp.where(cond, hi, lo)

    b1 = f > m(7)
    t2 = sel(b1, m(11), m(3))
    b2 = f > t2
    t3 = sel(b1, sel(b2, m(13), m(9)), sel(b2, m(5), m(1)))
    b3 = f > t3
    t4 = sel(b1,
             sel(b2, sel(b3, m(14), m(12)), sel(b3, m(10), m(8))),
             sel(b2, sel(b3, m(6), m(4)), sel(b3, m(2), m(0))))
    b4 = f > t4
    q = sel(b1,
            sel(b2,
                sel(b3, sel(b4, ctr(15), ctr(14)), sel(b4, ctr(13), ctr(12))),
                sel(b3, sel(b4, ctr(11), ctr(10)), sel(b4, ctr(9), ctr(8)))),
            sel(b2,
                sel(b3, sel(b4, ctr(7), ctr(6)), sel(b4, ctr(5), ctr(4))),
                sel(b3, sel(b4, ctr(3), ctr(2)), sel(b4, ctr(1), ctr(0)))))

    qs = jnp.where(even, jnp.roll(q, -1, axis=1), jnp.roll(q, 1, axis=1))
    o_ref[...] = (a * q + b * qs) * norm


def _tc_quant(x, rf, cents, bm, n_rows):
    B = x.shape[0]
    grid = (n_rows // bm,)
    return pl.pallas_call(
        _tc_body,
        grid=grid,
        in_specs=[
            pl.BlockSpec(memory_space=pltpu.SMEM),
            pl.BlockSpec((bm, _D), lambda i: (i, 0)),
            pl.BlockSpec((1, _D), lambda i: (0, 0)),
        ],
        out_specs=pl.BlockSpec((bm, _D), lambda i: (i, 0)),
        out_shape=jax.ShapeDtypeStruct((B, _D), jnp.float32),
    )(cents, x, rf)


_CH = 32

_GDN = lax.GatherDimensionNumbers(
    offset_dims=(), collapsed_slice_dims=(0,), start_index_map=(0,))


def _take16(v, idx):
    return lax.gather(v, idx[:, None], _GDN, slice_sizes=(1,),
                      mode=lax.GatherScatterMode.PROMISE_IN_BOUNDS)


def _sc_quant(x2d, rf, cents, n_rows, row0):
    info = plsc.get_sparse_core_info()
    NC, NS = info.num_cores, info.num_subcores
    NW = NC * NS
    rows_w = n_rows // NW
    n_ch = rows_w // _CH
    mesh = plsc.VectorSubcoreMesh(core_axis_name="c", subcore_axis_name="s")

    @functools.partial(
        pl.kernel,
        mesh=mesh,
        out_type=jax.ShapeDtypeStruct((n_rows, _D), jnp.float32),
        scratch_types=[
            pltpu.VMEM((_CH, _D), jnp.float32),
            pltpu.VMEM((_CH, _D), jnp.float32),
            pltpu.VMEM((_D,), jnp.float32),
            pltpu.VMEM((16,), jnp.float32),
        ],
    )
    def k(x_hbm, rf_hbm, cents_hbm, o_hbm, xbuf, obuf, rbuf, tbuf):
        wid = lax.axis_index("s") * NC + lax.axis_index("c")
        base = wid * rows_w
        pltpu.sync_copy(rf_hbm, rbuf)
        pltpu.sync_copy(cents_hbm, tbuf)
        lane = lax.iota(jnp.int32, 16)
        swap = lane ^ 1
        emask = (lane & 1) == 0
        cents_v = tbuf[...]
        mids_v = 0.5 * (cents_v
                        + _take16(cents_v, jnp.minimum(lane + 1, 15)))
        m7 = _take16(mids_v, jnp.full((16,), 7, jnp.int32))

        def chunk_body(g, _carry):
            rstart = base + g * _CH
            pltpu.sync_copy(x_hbm.at[pl.ds(row0 + rstart, _CH)], xbuf)

            def row_body(r, _c2):
                def ssq_body(g4, accs):
                    vs = [xbuf[r, pl.ds(g4 * 64 + j * 16, 16)]
                          for j in range(4)]
                    return tuple(accs[j] + vs[j] * vs[j] for j in range(4))

                accs = lax.fori_loop(
                    0, 4, ssq_body,
                    tuple(jnp.zeros((16,), jnp.float32) for _ in range(4)))
                acc = (accs[0] + accs[1]) + (accs[2] + accs[3])
                for sh in (8, 4, 2, 1):
                    acc = acc + _take16(acc, lane ^ sh)
                sv = jnp.maximum(acc, 1e-30)
                i32 = lax.bitcast_convert_type(sv, jnp.int32)
                y = lax.bitcast_convert_type(
                    jnp.int32(0x5F3759DF) - (i32 >> 1), jnp.float32)
                for _ in range(3):
                    y = y * (1.5 - 0.5 * sv * y * y)
                norm = jnp.maximum(sv * y, 1e-8)
                inv = 1.0 / norm

                def group_body(g4, _c3):
                    cs = [g4 * 64 + j * 16 for j in range(4)]
                    vl = [xbuf[r, pl.ds(c, 16)] for c in cs]
                    rl = [rbuf[pl.ds(c, 16)] for c in cs]
                    rw = [_take16(rl[j], swap) for j in range(4)]
                    al = [jnp.where(emask, rl[j], rw[j]) for j in range(4)]
                    bl = [jnp.where(emask, -rw[j], rl[j]) for j in range(4)]
                    fl = [(al[j] * vl[j] + bl[j] * _take16(vl[j], swap)) * inv
                          for j in range(4)]
                    il = [jnp.where(fl[j] > m7, 8, 0) for j in range(4)]
                    for bit in (4, 2, 1):
                        tm = [_take16(mids_v, il[j] | (bit - 1))
                              for j in range(4)]
                        il = [il[j] | jnp.where(fl[j] > tm[j], bit, 0)
                              for j in range(4)]
                    ql = [_take16(cents_v, il[j]) for j in range(4)]
                    ol = [(al[j] * ql[j] + bl[j] * _take16(ql[j], swap)) * norm
                          for j in range(4)]
                    for j in range(4):
                        obuf[r, pl.ds(cs[j], 16)] = ol[j]
                    return 0

                lax.fori_loop(0, 4, group_body, 0)
                return 0

            lax.fori_loop(0, _CH, row_body, 0)
            pltpu.sync_copy(obuf, o_hbm.at[pl.ds(rstart, _CH)])
            return 0

        lax.fori_loop(0, n_ch, chunk_body, 0)

    return k(x2d, rf, cents)


@jax.jit
def kernel(x, rot2, centroids):
    rf2 = jnp.reshape(rot2, (1, _D))
    n_sc = x.shape[0] - _TC_ROWS
    if n_sc <= 0:
        return _tc_quant(x, rf2, centroids, _TC_BM, x.shape[0])
    if _TC_ROWS == 0:
        return _sc_quant(x, jnp.reshape(rf2, (_D,)), centroids, n_sc, 0)

    tc_out = _tc_quant(x, rf2, centroids, _TC_BM, _TC_ROWS)
    sc_out = _sc_quant(x, jnp.reshape(rf2, (_D,)), centroids, n_sc, _TC_ROWS)
    return lax.dynamic_update_slice(tc_out, sc_out, (_TC_ROWS, 0))
